# baseline (speedup 1.0000x reference)
import jax
import jax.numpy as jnp
import numpy as np
from jax.experimental import pallas as pl

HID = 16
NUM_LAYERS = 4
TOP_K = 2
MAX_ROUTING = 2
SCORE_SCALE_C = 1.0
BATCH = 256
D_FLAT = HID * 16 * 16

F32 = jnp.float32


def _conv1_kernel(p_ref, w_ref, b_ref, o_ref):
    o_ref[...] = jax.nn.relu(
        jnp.dot(p_ref[...], w_ref[...], preferred_element_type=F32) + b_ref[...])


def _route_kernel(u4_ref, uf_ref, gate_ref, gb_ref, wem_ref, be_ref,
                  fin_ref, lg_ref):
    ub = u4_ref[...]
    logits = jnp.dot(uf_ref[...], gate_ref[...],
                     preferred_element_type=F32) + gb_ref[...]
    lg_ref[...] = logits
    m = jnp.max(logits, axis=1, keepdims=True)
    ew = jnp.exp(logits - m)
    w = ew / jnp.sum(ew, axis=1, keepdims=True)
    cols = [w[:, e:e + 1] for e in range(NUM_LAYERS)]
    wes = []
    sels = []
    for e in range(NUM_LAYERS):
        rank = jnp.zeros_like(cols[e])
        for j in range(NUM_LAYERS):
            if j == e:
                continue
            beats = (cols[j] > cols[e]) | ((cols[j] == cols[e]) & (j < e))
            rank = rank + jnp.where(beats, 1.0, 0.0)
        sels.append(rank < TOP_K)
    tsum = sum(jnp.where(sels[e], cols[e], 0.0) for e in range(NUM_LAYERS))
    for e in range(NUM_LAYERS):
        wes.append(jnp.where(sels[e], cols[e], 0.0) / tsum * SCORE_SCALE_C)
    up = jnp.pad(ub, ((0, 0), (1, 1), (1, 1), (0, 0)))
    parts = [up[:, dy:dy + 16, dx:dx + 16, :]
             for dy in range(3) for dx in range(3)]
    patches = jnp.concatenate(parts, axis=-1)
    bb = ub.shape[0]
    p2 = patches.reshape(bb * 256, 9 * HID)
    eo = jnp.dot(p2, wem_ref[...], preferred_element_type=F32) + be_ref[...]
    eo = jax.nn.relu(eo).reshape(bb, 256, NUM_LAYERS * HID)
    acc = eo[:, :, 0:HID] * wes[0][:, :, None]
    for e in range(1, NUM_LAYERS):
        acc = acc + eo[:, :, e * HID:(e + 1) * HID] * wes[e][:, :, None]
    fin_ref[...] = acc


def _mm_kernel(a_ref, w_ref, b_ref, o_ref):
    o_ref[...] = jnp.dot(a_ref[...], w_ref[...],
                         preferred_element_type=F32) + b_ref[...]


def _tail_kernel(p_ref, w_ref, b_ref, o_ref):
    o = jax.nn.relu(jnp.dot(p_ref[...], w_ref[...],
                            preferred_element_type=F32) + b_ref[...])
    o = o.reshape(BATCH * 16, 4, HID // 2)
    o_ref[...] = jnp.max(o, axis=1)


def _head_kernel(h_ref, w1_ref, b1_ref, w2_ref, b2_ref, o_ref):
    h = jax.nn.relu(jnp.dot(h_ref[...], w1_ref[...],
                            preferred_element_type=F32) + b1_ref[...])
    o_ref[...] = jnp.dot(h, w2_ref[...],
                         preferred_element_type=F32) + b2_ref[...]


def kernel(x, W1, b1, We, be, Wl, bl, fc1w, fc1b, fc2w, fc2b,
           gate_w, gate_b, tw, tb):
    idx = np.arange(D_FLAT)
    yy, xx, cc = idx // 256, (idx // 16) % 16, idx % 16
    q = (cc * 256 + yy * 16 + xx).astype(np.int32)
    gate_p = gate_w.T[q]
    twp = tw.T[q][:, q]
    tbp = tb[q][None, :]
    w1m = W1.reshape(HID, 27).T
    wem = jnp.transpose(We, (3, 4, 2, 0, 1)).reshape(9 * HID, NUM_LAYERS * HID)
    bef = be.reshape(1, NUM_LAYERS * HID)
    wlm = jnp.transpose(Wl, (2, 3, 1, 0)).reshape(9 * HID, HID // 2)
    mm = np.arange(128)
    r = ((mm % 8) * 16 + (mm // 32) * 4 + ((mm % 32) // 8)).astype(np.int32)
    fc1p = fc1w[:, r].T
    fc2p = fc2w.T

    xp = jnp.pad(x, ((0, 0), (0, 0), (1, 1), (1, 1)))
    sl = [xp[:, :, ky:ky + 32:2, kx:kx + 32:2]
          for ky in range(3) for kx in range(3)]
    pat = jnp.stack(sl, axis=2)
    pat = pat.transpose(0, 3, 4, 1, 2).reshape(BATCH * 256, 27)
    rb = BATCH * 256 // 8
    u = pl.pallas_call(
        _conv1_kernel,
        grid=(8,),
        in_specs=[pl.BlockSpec((rb, 27), lambda i: (i, 0)),
                  pl.BlockSpec((27, HID), lambda i: (0, 0)),
                  pl.BlockSpec((1, HID), lambda i: (0, 0))],
        out_specs=pl.BlockSpec((rb, HID), lambda i: (i, 0)),
        out_shape=jax.ShapeDtypeStruct((BATCH * 256, HID), F32),
    )(pat, w1m, b1[None, :])

    uf = u.reshape(BATCH, D_FLAT)
    u4 = u.reshape(BATCH, 16, 16, HID)

    logits_out = []
    bbk = 32
    for _ in range(MAX_ROUTING):
        fin, lg = pl.pallas_call(
            _route_kernel,
            grid=(BATCH // bbk,),
            in_specs=[
                pl.BlockSpec((bbk, 16, 16, HID), lambda i: (i, 0, 0, 0)),
                pl.BlockSpec((bbk, D_FLAT), lambda i: (i, 0)),
                pl.BlockSpec((D_FLAT, NUM_LAYERS), lambda i: (0, 0)),
                pl.BlockSpec((1, NUM_LAYERS), lambda i: (0, 0)),
                pl.BlockSpec((9 * HID, NUM_LAYERS * HID), lambda i: (0, 0)),
                pl.BlockSpec((1, NUM_LAYERS * HID), lambda i: (0, 0)),
            ],
            out_specs=[pl.BlockSpec((bbk, 256, HID), lambda i: (i, 0, 0)),
                       pl.BlockSpec((bbk, NUM_LAYERS), lambda i: (i, 0))],
            out_shape=[jax.ShapeDtypeStruct((BATCH, 256, HID), F32),
                       jax.ShapeDtypeStruct((BATCH, NUM_LAYERS), F32)],
        )(u4, uf, gate_p, gate_b[None, :], wem, bef)
        logits_out.append(lg)
        fin2 = fin.reshape(BATCH, D_FLAT)
        cbk = 512
        uf = pl.pallas_call(
            _mm_kernel,
            grid=(D_FLAT // cbk,),
            in_specs=[pl.BlockSpec((BATCH, D_FLAT), lambda j: (0, 0)),
                      pl.BlockSpec((D_FLAT, cbk), lambda j: (0, j)),
                      pl.BlockSpec((1, cbk), lambda j: (0, j))],
            out_specs=pl.BlockSpec((BATCH, cbk), lambda j: (0, j)),
            out_shape=jax.ShapeDtypeStruct((BATCH, D_FLAT), F32),
        )(fin2, twp, tbp)
        u4 = uf.reshape(BATCH, 16, 16, HID)

    up = jnp.pad(u4, ((0, 0), (1, 1), (1, 1), (0, 0)))
    sl = [up[:, ky:ky + 16:2, kx:kx + 16:2, :]
          for ky in range(3) for kx in range(3)]
    patl = jnp.concatenate(sl, axis=-1)
    patl = patl.reshape(BATCH, 4, 2, 4, 2, 9 * HID)
    patl = patl.transpose(0, 1, 3, 2, 4, 5).reshape(BATCH * 64, 9 * HID)
    pooled = pl.pallas_call(
        _tail_kernel,
        in_specs=[pl.BlockSpec((BATCH * 64, 9 * HID), lambda: (0, 0)),
                  pl.BlockSpec((9 * HID, HID // 2), lambda: (0, 0)),
                  pl.BlockSpec((1, HID // 2), lambda: (0, 0))],
        out_specs=pl.BlockSpec((BATCH * 16, HID // 2), lambda: (0, 0)),
        out_shape=jax.ShapeDtypeStruct((BATCH * 16, HID // 2), F32),
    )(patl, wlm, bl[None, :])

    h0 = pooled.reshape(BATCH, 128)
    out = pl.pallas_call(
        _head_kernel,
        in_specs=[pl.BlockSpec((BATCH, 128), lambda: (0, 0)),
                  pl.BlockSpec((128, HID), lambda: (0, 0)),
                  pl.BlockSpec((1, HID), lambda: (0, 0)),
                  pl.BlockSpec((HID, 10), lambda: (0, 0)),
                  pl.BlockSpec((1, 10), lambda: (0, 0))],
        out_specs=pl.BlockSpec((BATCH, 10), lambda: (0, 0)),
        out_shape=jax.ShapeDtypeStruct((BATCH, 10), F32),
    )(h0, fc1p, fc1b[None, :], fc2p, fc2b[None, :])

    return (out, (logits_out[0], logits_out[1]))


# speedup vs baseline: 1.2802x; 1.2802x over previous
import jax
import jax.numpy as jnp
import numpy as np
from jax.experimental import pallas as pl

HID = 16
NUM_LAYERS = 4
TOP_K = 2
MAX_ROUTING = 2
SCORE_SCALE_C = 1.0
BATCH = 256
D_FLAT = HID * 16 * 16

F32 = jnp.float32


def _conv1_kernel(p_ref, w_ref, b_ref, o_ref):
    o_ref[...] = jax.nn.relu(
        jnp.dot(p_ref[...], w_ref[...], preferred_element_type=F32) + b_ref[...])


def _route_kernel(u4_ref, uf_ref, gate_ref, gb_ref, wem_ref, be_ref,
                  fin_ref, lg_ref):
    ub = u4_ref[...]
    logits = jnp.dot(uf_ref[...], gate_ref[...],
                     preferred_element_type=F32) + gb_ref[...]
    lg_ref[...] = logits
    m = jnp.max(logits, axis=1, keepdims=True)
    ew = jnp.exp(logits - m)
    w = ew / jnp.sum(ew, axis=1, keepdims=True)
    cols = [w[:, e:e + 1] for e in range(NUM_LAYERS)]
    sels = []
    for e in range(NUM_LAYERS):
        rank = jnp.zeros_like(cols[e])
        for j in range(NUM_LAYERS):
            if j == e:
                continue
            beats = (cols[j] > cols[e]) | ((cols[j] == cols[e]) & (j < e))
            rank = rank + jnp.where(beats, 1.0, 0.0)
        sels.append(rank < TOP_K)
    tsum = sum(jnp.where(sels[e], cols[e], 0.0) for e in range(NUM_LAYERS))
    wes = [jnp.where(sels[e], cols[e], 0.0) / tsum * SCORE_SCALE_C
           for e in range(NUM_LAYERS)]
    up = jnp.pad(ub, ((0, 0), (1, 1), (1, 1), (0, 0)))
    parts = [up[:, dy:dy + 16, dx:dx + 16, :]
             for dy in range(3) for dx in range(3)]
    patches = jnp.concatenate(parts, axis=-1)
    bb = ub.shape[0]
    p2 = patches.reshape(bb * 256, 9 * HID)
    eo = jnp.dot(p2, wem_ref[...], preferred_element_type=F32) + be_ref[...]
    eo = jax.nn.relu(eo).reshape(bb, 256, NUM_LAYERS * HID)
    acc = eo[:, :, 0:HID] * wes[0][:, :, None]
    for e in range(1, NUM_LAYERS):
        acc = acc + eo[:, :, e * HID:(e + 1) * HID] * wes[e][:, :, None]
    fin_ref[...] = acc


def _mmt_kernel(w_ref, a_ref, b_ref, o_ref):
    o_ref[...] = jnp.dot(w_ref[...], a_ref[...],
                         preferred_element_type=F32) + b_ref[...]


def _tail_kernel(p_ref, w_ref, b_ref, o_ref):
    o = jax.nn.relu(jnp.dot(p_ref[...], w_ref[...],
                            preferred_element_type=F32) + b_ref[...])
    o = o.reshape(BATCH * 16, 4, HID // 2)
    o_ref[...] = jnp.max(o, axis=1)


def _head_kernel(h_ref, w1_ref, b1_ref, w2_ref, b2_ref, o_ref):
    h = jax.nn.relu(jnp.dot(h_ref[...], w1_ref[...],
                            preferred_element_type=F32) + b1_ref[...])
    o_ref[...] = jnp.dot(h, w2_ref[...],
                         preferred_element_type=F32) + b2_ref[...]


def kernel(x, W1, b1, We, be, Wl, bl, fc1w, fc1b, fc2w, fc2b,
           gate_w, gate_b, tw, tb):
    w1m = W1.reshape(HID, 27).T
    wem = jnp.transpose(We, (3, 4, 2, 0, 1)).reshape(9 * HID, NUM_LAYERS * HID)
    bef = be.reshape(1, NUM_LAYERS * HID)
    wlm = jnp.transpose(Wl, (2, 3, 1, 0)).reshape(9 * HID, HID // 2)
    gtp = gate_w.T
    tbc = tb[:, None]

    xp = jnp.pad(x, ((0, 0), (0, 0), (1, 1), (1, 1)))
    sl = [xp[:, :, ky:ky + 32:2, kx:kx + 32:2]
          for ky in range(3) for kx in range(3)]
    pat = jnp.stack(sl, axis=2)
    pat = pat.transpose(0, 3, 4, 1, 2).reshape(BATCH * 256, 27)
    rb = BATCH * 256 // 8
    u = pl.pallas_call(
        _conv1_kernel,
        grid=(8,),
        in_specs=[pl.BlockSpec((rb, 27), lambda i: (i, 0)),
                  pl.BlockSpec((27, HID), lambda i: (0, 0)),
                  pl.BlockSpec((1, HID), lambda i: (0, 0))],
        out_specs=pl.BlockSpec((rb, HID), lambda i: (i, 0)),
        out_shape=jax.ShapeDtypeStruct((BATCH * 256, HID), F32),
    )(pat, w1m, b1[None, :])

    u4 = u.reshape(BATCH, 16, 16, HID)

    logits_out = []
    bbk = 32
    for _ in range(MAX_ROUTING):
        uf = u4.transpose(0, 3, 1, 2).reshape(BATCH, D_FLAT)
        fin, lg = pl.pallas_call(
            _route_kernel,
            grid=(BATCH // bbk,),
            in_specs=[
                pl.BlockSpec((bbk, 16, 16, HID), lambda i: (i, 0, 0, 0)),
                pl.BlockSpec((bbk, D_FLAT), lambda i: (i, 0)),
                pl.BlockSpec((D_FLAT, NUM_LAYERS), lambda i: (0, 0)),
                pl.BlockSpec((1, NUM_LAYERS), lambda i: (0, 0)),
                pl.BlockSpec((9 * HID, NUM_LAYERS * HID), lambda i: (0, 0)),
                pl.BlockSpec((1, NUM_LAYERS * HID), lambda i: (0, 0)),
            ],
            out_specs=[pl.BlockSpec((bbk, 256, HID), lambda i: (i, 0, 0)),
                       pl.BlockSpec((bbk, NUM_LAYERS), lambda i: (i, 0))],
            out_shape=[jax.ShapeDtypeStruct((BATCH, 256, HID), F32),
                       jax.ShapeDtypeStruct((BATCH, NUM_LAYERS), F32)],
        )(u4, uf, gtp, gate_b[None, :], wem, bef)
        logits_out.append(lg)
        fint = fin.reshape(BATCH, 16, 16, HID).transpose(3, 1, 2, 0)
        fint = fint.reshape(D_FLAT, BATCH)
        rbk = 512
        outt = pl.pallas_call(
            _mmt_kernel,
            grid=(D_FLAT // rbk,),
            in_specs=[pl.BlockSpec((rbk, D_FLAT), lambda j: (j, 0)),
                      pl.BlockSpec((D_FLAT, BATCH), lambda j: (0, 0)),
                      pl.BlockSpec((rbk, 1), lambda j: (j, 0))],
            out_specs=pl.BlockSpec((rbk, BATCH), lambda j: (j, 0)),
            out_shape=jax.ShapeDtypeStruct((D_FLAT, BATCH), F32),
        )(tw, fint, tbc)
        u4 = outt.reshape(HID, 16, 16, BATCH).transpose(3, 1, 2, 0)

    up = jnp.pad(u4, ((0, 0), (1, 1), (1, 1), (0, 0)))
    sl = [up[:, ky:ky + 16:2, kx:kx + 16:2, :]
          for ky in range(3) for kx in range(3)]
    patl = jnp.concatenate(sl, axis=-1)
    patl = patl.reshape(BATCH, 4, 2, 4, 2, 9 * HID)
    patl = patl.transpose(0, 1, 3, 2, 4, 5).reshape(BATCH * 64, 9 * HID)
    pooled = pl.pallas_call(
        _tail_kernel,
        in_specs=[pl.BlockSpec((BATCH * 64, 9 * HID), lambda: (0, 0)),
                  pl.BlockSpec((9 * HID, HID // 2), lambda: (0, 0)),
                  pl.BlockSpec((1, HID // 2), lambda: (0, 0))],
        out_specs=pl.BlockSpec((BATCH * 16, HID // 2), lambda: (0, 0)),
        out_shape=jax.ShapeDtypeStruct((BATCH * 16, HID // 2), F32),
    )(patl, wlm, bl[None, :])

    h0 = pooled.reshape(BATCH, 16, HID // 2).transpose(0, 2, 1)
    h0 = h0.reshape(BATCH, 128)
    out = pl.pallas_call(
        _head_kernel,
        in_specs=[pl.BlockSpec((BATCH, 128), lambda: (0, 0)),
                  pl.BlockSpec((128, HID), lambda: (0, 0)),
                  pl.BlockSpec((1, HID), lambda: (0, 0)),
                  pl.BlockSpec((HID, 10), lambda: (0, 0)),
                  pl.BlockSpec((1, 10), lambda: (0, 0))],
        out_specs=pl.BlockSpec((BATCH, 10), lambda: (0, 0)),
        out_shape=jax.ShapeDtypeStruct((BATCH, 10), F32),
    )(h0, fc1w.T, fc1b[None, :], fc2w.T, fc2b[None, :])

    return (out, (logits_out[0], logits_out[1]))
